# Initial kernel scaffold; baseline (speedup 1.0000x reference)
import jax
import jax.numpy as jnp
from jax.experimental import pallas as pl


def kernel(x, w1, b1, w2, b2, w3, b3, seed):
    raise NotImplementedError("write your pallas kernel here")


# speedup vs baseline: 1.8926x; 1.8926x over previous
import jax
import jax.numpy as jnp
from jax.experimental import pallas as pl
from jax.experimental.pallas import tpu as pltpu

_P = 0.2
_THRESH = -(2 ** 31) + int(round(_P * (2 ** 32)))
_SCALE = 1.0 / (1.0 - _P)


def _srl(v, k):
    return jnp.bitwise_and(jnp.right_shift(v, k), (1 << (32 - k)) - 1)


def _mix(z):
    z = z ^ _srl(z, 16)
    z = z * jnp.int32(0x7FEB352D)
    z = z ^ _srl(z, 15)
    z = z * jnp.int32(-2073777525)
    z = z ^ _srl(z, 16)
    return z


def _mlp_kernel(seed_ref, x_ref, w1_ref, b1_ref, w2_ref, b2_ref, w3_ref, b3_ref,
                o_ref):
    x = x_ref[...]
    h1 = jnp.dot(w1_ref[...], x, preferred_element_type=jnp.float32)
    h1 = jnp.maximum(h1 + b1_ref[...], 0.0)
    h2 = jnp.dot(w2_ref[...], h1, preferred_element_type=jnp.float32)
    h2 = jnp.maximum(h2 + b2_ref[...], 0.0)

    tile_b = h2.shape[1]
    col = pl.program_id(0) * tile_b + jax.lax.broadcasted_iota(jnp.int32, h2.shape, 1)
    row = jax.lax.broadcasted_iota(jnp.int32, h2.shape, 0)
    z = seed_ref[0] + col * jnp.int32(-1640531527) + row * jnp.int32(-2049221013)
    z = _mix(z)
    h2 = jnp.where(z >= jnp.int32(_THRESH), h2 * jnp.float32(_SCALE), 0.0)

    out = jnp.dot(w3_ref[...], h2, preferred_element_type=jnp.float32)
    o_ref[...] = (out + b3_ref[...]).astype(o_ref.dtype)


def kernel(x, w1, b1, w2, b2, w3, b3, seed):
    B = x.shape[0]
    tile_b = 2048
    assert B % tile_b == 0
    grid = B // tile_b

    x_t = x.T
    seed_arr = jnp.full((1,), seed, dtype=jnp.int32)

    def full(shape):
        return pl.BlockSpec(shape, lambda b, s: (0, 0))

    out_t = pl.pallas_call(
        _mlp_kernel,
        out_shape=jax.ShapeDtypeStruct((3, B), jnp.float32),
        grid_spec=pltpu.PrefetchScalarGridSpec(
            num_scalar_prefetch=1,
            grid=(grid,),
            in_specs=[
                pl.BlockSpec((4, tile_b), lambda b, s: (0, b)),
                full(w1.shape), full(b1.shape),
                full(w2.shape), full(b2.shape),
                full(w3.shape), full(b3.shape),
            ],
            out_specs=pl.BlockSpec((3, tile_b), lambda b, s: (0, b)),
        ),
        compiler_params=pltpu.CompilerParams(
            dimension_semantics=("parallel",),
        ),
    )(seed_arr, x_t, w1, b1, w2, b2, w3, b3)

    return out_t[:, :B].T


# speedup vs baseline: 5.9333x; 3.1349x over previous
import jax
import jax.numpy as jnp
from jax.experimental import pallas as pl
from jax.experimental.pallas import tpu as pltpu

_P = 0.2
_THRESH = -(2 ** 31) + int(round(_P * (2 ** 32)))
_SCALE = 1.0 / (1.0 - _P)


def _srl(v, k):
    return jnp.bitwise_and(jnp.right_shift(v, k), (1 << (32 - k)) - 1)


def _mix(z):
    z = z ^ _srl(z, 16)
    z = z * jnp.int32(0x7FEB352D)
    z = z ^ _srl(z, 15)
    z = z * jnp.int32(-2073777525)
    z = z ^ _srl(z, 16)
    return z


def _mlp_kernel(seed_ref, x_ref, w1_ref, b1_ref, w2_ref, b2_ref, w3_ref, b3_ref,
                o_ref):
    x = x_ref[...]
    h1 = jnp.dot(w1_ref[...], x, preferred_element_type=jnp.float32)
    h1 = jnp.maximum(h1 + b1_ref[...], 0.0)
    h2 = jnp.dot(w2_ref[...], h1, preferred_element_type=jnp.float32)
    h2 = jnp.maximum(h2 + b2_ref[...], 0.0)

    tile_b = h2.shape[1]
    col = pl.program_id(0) * tile_b + jax.lax.broadcasted_iota(jnp.int32, h2.shape, 1)
    row = jax.lax.broadcasted_iota(jnp.int32, h2.shape, 0)
    z = seed_ref[0] + col * jnp.int32(-1640531527) + row * jnp.int32(-2049221013)
    z = _mix(z)
    h2 = jnp.where(z >= jnp.int32(_THRESH), h2 * jnp.float32(_SCALE), 0.0)

    out = jnp.dot(w3_ref[...], h2, preferred_element_type=jnp.float32)
    o_ref[...] = (out + b3_ref[...]).astype(o_ref.dtype)


def kernel(x, w1, b1, w2, b2, w3, b3, seed):
    B = x.shape[0]
    tile_b = 8192
    assert B % tile_b == 0
    grid = B // tile_b

    x_t = x.T
    seed_arr = jnp.full((1,), seed, dtype=jnp.int32)

    def full(shape):
        return pl.BlockSpec(shape, lambda b, s: (0, 0))

    out_t = pl.pallas_call(
        _mlp_kernel,
        out_shape=jax.ShapeDtypeStruct((3, B), jnp.float32),
        grid_spec=pltpu.PrefetchScalarGridSpec(
            num_scalar_prefetch=1,
            grid=(grid,),
            in_specs=[
                pl.BlockSpec((4, tile_b), lambda b, s: (0, b)),
                full(w1.shape), full(b1.shape),
                full(w2.shape), full(b2.shape),
                full(w3.shape), full(b3.shape),
            ],
            out_specs=pl.BlockSpec((3, tile_b), lambda b, s: (0, b)),
        ),
        compiler_params=pltpu.CompilerParams(
            dimension_semantics=("parallel",),
        ),
    )(seed_arr, x_t, w1, b1, w2, b2, w3, b3)

    return out_t.T


# speedup vs baseline: 11.1682x; 1.8823x over previous
import jax
import jax.numpy as jnp
from jax.experimental import pallas as pl
from jax.experimental.pallas import tpu as pltpu

_P = 0.2
_THRESH = -(2 ** 31) + int(round(_P * (2 ** 32)))
_SCALE = 1.0 / (1.0 - _P)


def _srl(v, k):
    return jnp.bitwise_and(jnp.right_shift(v, k), (1 << (32 - k)) - 1)


def _mix(z):
    z = z ^ _srl(z, 16)
    z = z * jnp.int32(0x7FEB352D)
    z = z ^ _srl(z, 15)
    z = z * jnp.int32(-2073777525)
    z = z ^ _srl(z, 16)
    return z


def _mlp_kernel(seed_ref, x_ref, w1_ref, b1_ref, w2_ref, b2_ref, w3_ref, b3_ref,
                o_ref):
    x = x_ref[...]
    h1 = jnp.dot(w1_ref[...], x, preferred_element_type=jnp.float32)
    h1 = jnp.maximum(h1 + b1_ref[...], 0.0)
    h2 = jnp.dot(w2_ref[...], h1, preferred_element_type=jnp.float32)
    h2 = jnp.maximum(h2 + b2_ref[...], 0.0)

    tile_b = h2.shape[1]
    col = pl.program_id(0) * tile_b + jax.lax.broadcasted_iota(jnp.int32, h2.shape, 1)
    row = jax.lax.broadcasted_iota(jnp.int32, h2.shape, 0)
    z = seed_ref[0] + col * jnp.int32(-1640531527) + row * jnp.int32(-2049221013)
    z = _mix(z)
    h2 = jnp.where(z >= jnp.int32(_THRESH), h2 * jnp.float32(_SCALE), 0.0)

    out = jnp.dot(w3_ref[...], h2, preferred_element_type=jnp.float32)
    o_ref[...] = (out + b3_ref[...]).astype(o_ref.dtype)


def kernel(x, w1, b1, w2, b2, w3, b3, seed):
    B = x.shape[0]
    tile_b = 32768
    assert B % tile_b == 0
    grid = B // tile_b

    x_t = x.T
    seed_arr = jnp.full((1,), seed, dtype=jnp.int32)

    def full(shape):
        return pl.BlockSpec(shape, lambda b, s: (0, 0))

    out_t = pl.pallas_call(
        _mlp_kernel,
        out_shape=jax.ShapeDtypeStruct((3, B), jnp.float32),
        grid_spec=pltpu.PrefetchScalarGridSpec(
            num_scalar_prefetch=1,
            grid=(grid,),
            in_specs=[
                pl.BlockSpec((4, tile_b), lambda b, s: (0, b)),
                full(w1.shape), full(b1.shape),
                full(w2.shape), full(b2.shape),
                full(w3.shape), full(b3.shape),
            ],
            out_specs=pl.BlockSpec((3, tile_b), lambda b, s: (0, b)),
        ),
        compiler_params=pltpu.CompilerParams(
            dimension_semantics=("parallel",),
        ),
    )(seed_arr, x_t, w1, b1, w2, b2, w3, b3)

    return out_t.T


# speedup vs baseline: 12.4367x; 1.1136x over previous
import jax
import jax.numpy as jnp
from jax.experimental import pallas as pl
from jax.experimental.pallas import tpu as pltpu

_P = 0.2
_THRESH = -(2 ** 31) + int(round(_P * (2 ** 32)))
_SCALE = 1.0 / (1.0 - _P)


def _srl(v, k):
    return jnp.bitwise_and(jnp.right_shift(v, k), (1 << (32 - k)) - 1)


def _mix(z):
    z = z ^ _srl(z, 16)
    z = z * jnp.int32(0x7FEB352D)
    z = z ^ _srl(z, 15)
    z = z * jnp.int32(-2073777525)
    z = z ^ _srl(z, 16)
    return z


def _mlp_kernel(seed_ref, x_ref, w1_ref, b1_ref, w2_ref, b2_ref, w3_ref, b3_ref,
                o_ref):
    x = x_ref[...]
    h1 = jnp.dot(w1_ref[...], x, preferred_element_type=jnp.float32)
    h1 = jnp.maximum(h1 + b1_ref[...], 0.0)
    h2 = jnp.dot(w2_ref[...], h1, preferred_element_type=jnp.float32)
    h2 = jnp.maximum(h2 + b2_ref[...], 0.0)

    tile_b = h2.shape[1]
    col = pl.program_id(0) * tile_b + jax.lax.broadcasted_iota(jnp.int32, h2.shape, 1)
    row = jax.lax.broadcasted_iota(jnp.int32, h2.shape, 0)
    z = seed_ref[0] + col * jnp.int32(-1640531527) + row * jnp.int32(-2049221013)
    z = _mix(z)
    h2 = jnp.where(z >= jnp.int32(_THRESH), h2 * jnp.float32(_SCALE), 0.0)

    out = jnp.dot(w3_ref[...], h2, preferred_element_type=jnp.float32)
    o_ref[...] = (out + b3_ref[...]).astype(o_ref.dtype)


def kernel(x, w1, b1, w2, b2, w3, b3, seed):
    B = x.shape[0]
    tile_b = 131072
    assert B % tile_b == 0
    grid = B // tile_b

    x_t = x.T
    seed_arr = jnp.full((1,), seed, dtype=jnp.int32)

    def full(shape):
        return pl.BlockSpec(shape, lambda b, s: (0, 0))

    out_t = pl.pallas_call(
        _mlp_kernel,
        out_shape=jax.ShapeDtypeStruct((3, B), jnp.float32),
        grid_spec=pltpu.PrefetchScalarGridSpec(
            num_scalar_prefetch=1,
            grid=(grid,),
            in_specs=[
                pl.BlockSpec((4, tile_b), lambda b, s: (0, b)),
                full(w1.shape), full(b1.shape),
                full(w2.shape), full(b2.shape),
                full(w3.shape), full(b3.shape),
            ],
            out_specs=pl.BlockSpec((3, tile_b), lambda b, s: (0, b)),
        ),
        compiler_params=pltpu.CompilerParams(
            dimension_semantics=("parallel",),
        ),
    )(seed_arr, x_t, w1, b1, w2, b2, w3, b3)

    return out_t.T
